# baseline (device time: 32627 ns/iter reference)
import functools

import jax
import jax.numpy as jnp
import numpy as np
from jax import lax
from jax.experimental import pallas as pl
from jax.experimental.pallas import tpu as pltpu

N_DEV = 4
DH = 64


def _ring_allreduce(partial):
    M, N = partial.shape

    def body(p_ref, out_ref, comm_ref, send_sems, recv_sems):
        my = lax.axis_index("i")
        left = lax.rem(my + N_DEV - 1, N_DEV)
        right = lax.rem(my + 1, N_DEV)

        barrier_sem = pltpu.get_barrier_semaphore()
        for nbr in [left, right]:
            pl.semaphore_signal(
                barrier_sem, inc=1,
                device_id=(nbr,), device_id_type=pl.DeviceIdType.MESH,
            )
        pl.semaphore_wait(barrier_sem, 2)

        comm_ref[0] = p_ref[...]

        for h in range(N_DEV - 1):
            rdma = pltpu.make_async_remote_copy(
                src_ref=comm_ref.at[h],
                dst_ref=comm_ref.at[h + 1],
                send_sem=send_sems.at[h],
                recv_sem=recv_sems.at[h],
                device_id=(right,),
                device_id_type=pl.DeviceIdType.MESH,
            )
            rdma.start()
            rdma.wait()

        out_ref[...] = (
            (comm_ref[0] + comm_ref[1]) + (comm_ref[2] + comm_ref[3])
        )

    return pl.pallas_call(
        body,
        out_shape=jax.ShapeDtypeStruct((M, N), jnp.float32),
        in_specs=[pl.BlockSpec(memory_space=pltpu.VMEM)],
        out_specs=pl.BlockSpec(memory_space=pltpu.VMEM),
        scratch_shapes=[
            pltpu.VMEM((N_DEV, M, N), jnp.float32),
            pltpu.SemaphoreType.DMA((N_DEV - 1,)),
            pltpu.SemaphoreType.DMA((N_DEV - 1,)),
        ],
        compiler_params=pltpu.CompilerParams(collective_id=0),
    )(partial)


def kernel(x, Wq, Wk, Wv, Wo):
    B, Sq, D = x.shape
    Hl = Wq.shape[1] // DH

    xf = x.reshape(B * Sq, D)
    q = (xf @ Wq).reshape(B, Sq, Hl, DH)
    k = (xf @ Wk).reshape(B, Sq, Hl, DH)
    v = (xf @ Wv).reshape(B, Sq, Hl, DH)

    inv = 1.0 / (10000.0 ** (np.arange(0, DH, 2) / DH))
    pos = np.arange(Sq)[:, None] * inv[None, :]
    cos = jnp.asarray(np.repeat(np.cos(pos), 2, axis=-1).astype(np.float32))
    sin = jnp.asarray(np.repeat(np.sin(pos), 2, axis=-1).astype(np.float32))
    cos = cos[None, :, None, :]
    sin = sin[None, :, None, :]

    def rot(t):
        t2 = t.reshape(B, Sq, Hl, DH // 2, 2)
        t_r = jnp.stack([-t2[..., 1], t2[..., 0]], axis=-1).reshape(B, Sq, Hl, DH)
        return t * cos + t_r * sin

    Q = rot(q)
    K = rot(k)
    s = jnp.einsum("bihd,bjhd->bhij", Q, K) * 0.125
    s_max = jnp.max(s, axis=-1, keepdims=True)
    w = jnp.exp(s - s_max)
    w = w / jnp.sum(w, axis=-1, keepdims=True)
    ctx = jnp.einsum("bhij,bjhd->bihd", w, v).reshape(B * Sq, Hl * DH)

    partial = ctx @ Wo
    out = _ring_allreduce(partial)
    return out.reshape(B, Sq, D)


# device time: 19475 ns/iter; 1.6753x vs baseline; 1.6753x over previous
import functools

import jax
import jax.numpy as jnp
import numpy as np
from jax import lax
from jax.experimental import pallas as pl
from jax.experimental.pallas import tpu as pltpu

N_DEV = 4
DH = 64


def _allreduce_2phase(pL, pR):
    M, H = pL.shape

    def body(pL_ref, pR_ref, out_ref, bufs, send_sems, recv_sems):
        my = lax.axis_index("i")
        pA = my ^ 1
        pB = 3 - my

        barrier_sem = pltpu.get_barrier_semaphore()
        for nbr in [pA, pB]:
            pl.semaphore_signal(
                barrier_sem, inc=1,
                device_id=(nbr,), device_id_type=pl.DeviceIdType.MESH,
            )
        pl.semaphore_wait(barrier_sem, 2)

        l1 = pltpu.make_async_remote_copy(
            src_ref=pL_ref, dst_ref=bufs.at[0],
            send_sem=send_sems.at[0], recv_sem=recv_sems.at[0],
            device_id=(pA,), device_id_type=pl.DeviceIdType.MESH,
        )
        r1 = pltpu.make_async_remote_copy(
            src_ref=pR_ref, dst_ref=bufs.at[1],
            send_sem=send_sems.at[1], recv_sem=recv_sems.at[1],
            device_id=(pB,), device_id_type=pl.DeviceIdType.MESH,
        )
        l1.start()
        r1.start()
        l1.wait()
        bufs[2] = pL_ref[...] + bufs[0]
        r1.wait()
        bufs[3] = pR_ref[...] + bufs[1]

        l2 = pltpu.make_async_remote_copy(
            src_ref=bufs.at[2], dst_ref=bufs.at[4],
            send_sem=send_sems.at[2], recv_sem=recv_sems.at[2],
            device_id=(pB,), device_id_type=pl.DeviceIdType.MESH,
        )
        r2 = pltpu.make_async_remote_copy(
            src_ref=bufs.at[3], dst_ref=bufs.at[5],
            send_sem=send_sems.at[3], recv_sem=recv_sems.at[3],
            device_id=(pA,), device_id_type=pl.DeviceIdType.MESH,
        )
        l2.start()
        r2.start()
        l2.wait()
        out_ref[:, :H] = bufs[2] + bufs[4]
        r2.wait()
        out_ref[:, H:] = bufs[3] + bufs[5]

    return pl.pallas_call(
        body,
        out_shape=jax.ShapeDtypeStruct((M, 2 * H), jnp.float32),
        in_specs=[
            pl.BlockSpec(memory_space=pltpu.VMEM),
            pl.BlockSpec(memory_space=pltpu.VMEM),
        ],
        out_specs=pl.BlockSpec(memory_space=pltpu.VMEM),
        scratch_shapes=[
            pltpu.VMEM((6, M, H), jnp.float32),
            pltpu.SemaphoreType.DMA((4,)),
            pltpu.SemaphoreType.DMA((4,)),
        ],
        compiler_params=pltpu.CompilerParams(collective_id=0),
    )(pL, pR)


def kernel(x, Wq, Wk, Wv, Wo):
    B, Sq, D = x.shape
    Hl = Wq.shape[1] // DH

    xf = x.reshape(B * Sq, D)
    q = (xf @ Wq).reshape(B, Sq, Hl, DH)
    k = (xf @ Wk).reshape(B, Sq, Hl, DH)
    v = (xf @ Wv).reshape(B, Sq, Hl, DH)

    inv = 1.0 / (10000.0 ** (np.arange(0, DH, 2) / DH))
    pos = np.arange(Sq)[:, None] * inv[None, :]
    cos = jnp.asarray(np.repeat(np.cos(pos), 2, axis=-1).astype(np.float32))
    sin = jnp.asarray(np.repeat(np.sin(pos), 2, axis=-1).astype(np.float32))
    cos = cos[None, :, None, :]
    sin = sin[None, :, None, :]

    def rot(t):
        t2 = t.reshape(B, Sq, Hl, DH // 2, 2)
        t_r = jnp.stack([-t2[..., 1], t2[..., 0]], axis=-1).reshape(B, Sq, Hl, DH)
        return t * cos + t_r * sin

    Q = rot(q)
    K = rot(k)
    s = jnp.einsum("bihd,bjhd->bhij", Q, K) * 0.125
    s_max = jnp.max(s, axis=-1, keepdims=True)
    w = jnp.exp(s - s_max)
    w = w / jnp.sum(w, axis=-1, keepdims=True)
    ctx = jnp.einsum("bhij,bjhd->bihd", w, v).reshape(B * Sq, Hl * DH)

    H = D // 2
    pL = ctx @ Wo[:, :H]
    pR = ctx @ Wo[:, H:]
    out = _allreduce_2phase(pL, pR)
    return out.reshape(B, Sq, D)
